# baseline (speedup 1.0000x reference)
import jax
import jax.numpy as jnp
from jax.experimental import pallas as pl
from jax.experimental.pallas import tpu as pltpu

_N = 4096
_F = 128
_C = 64
_BV = 512
_NV = _N // _BV

_DN = (((1,), (0,)), ((), ()))


def _mm(a, b):
    return jax.lax.dot_general(a, b, _DN, preferred_element_type=jnp.float32)


def _body(hcatT_ref, adj_ref, w0, b0, w1, b1, w12, b12, ws2, b2,
          out_ref, adj_scr, ideg_scr, x1_scr, x2_scr, y2_scr):
    s = pl.program_id(0)
    v = pl.program_id(1)
    cols = pl.ds(v * _BV, _BV)

    @pl.when(s == 0)
    def _stage0():
        ab = adj_ref[...].astype(jnp.bfloat16)
        adj_scr[:, cols] = ab
        zT = _mm(hcatT_ref[...], ab)
        ideg = 1.0 / jnp.maximum(zT[_F:_F + 1, :], 1.0)
        ideg_scr[:, cols] = ideg
        zs = (zT[:_F, :] * ideg).astype(jnp.bfloat16)
        cat = jnp.concatenate([hcatT_ref[:_F, cols], zs], axis=0)
        x1T = _mm(w0[...], cat) + b0[...]
        x1_scr[:, cols] = x1T.astype(jnp.bfloat16)

    @pl.when(s == 1)
    def _stage1():
        ab = adj_scr[:, cols]
        zT = _mm(x1_scr[...], ab)
        zs = (zT * ideg_scr[:, cols]).astype(jnp.bfloat16)
        cat = jnp.concatenate([x1_scr[:, cols], zs], axis=0)
        x2T = _mm(w1[...], cat) + b1[...]
        x2_scr[:, cols] = x2T.astype(jnp.bfloat16)
        y2T = _mm(w12[...], cat) + b12[...]
        y2_scr[:, cols] = y2T.astype(jnp.bfloat16)

    @pl.when(s == 2)
    def _stage2():
        ab = adj_scr[:, cols]
        zT = _mm(y2_scr[...], ab)
        zs = zT * ideg_scr[:, cols]
        out_ref[...] = _mm(ws2[...], x2_scr[:, cols]) + zs + b2[...]


def kernel(h, adj, W_self0, W_neigh0, b0, W_self1, W_neigh1, b1,
           W_self2, W_neigh2, b2):
    bf = jnp.bfloat16
    hcatT = jnp.concatenate(
        [h.T.astype(bf), jnp.ones((8, _N), bf)], axis=0)
    w0 = jnp.concatenate([W_self0, W_neigh0], axis=1).astype(bf)
    w1cat = jnp.concatenate([W_self1, W_neigh1], axis=1)
    w1 = w1cat.astype(bf)
    w12 = (W_neigh2 @ w1cat).astype(bf)
    b12 = (W_neigh2 @ b1).reshape(-1, 1)
    full = lambda shape: pl.BlockSpec(shape, lambda s, v: (0, 0))
    outT = pl.pallas_call(
        _body,
        grid=(3, _NV),
        in_specs=[
            full((_F + 8, _N)),
            pl.BlockSpec((_N, _BV),
                         lambda s, v: (0, jnp.where(s == 0, v, _NV - 1))),
            full((_F, 2 * _F)), full((_F, 1)),
            full((_F, 2 * _F)), full((_F, 1)),
            full((_C, 2 * _F)), full((_C, 1)),
            full((_C, _F)), full((_C, 1)),
        ],
        out_specs=pl.BlockSpec((_C, _BV),
                               lambda s, v: (0, jnp.where(s == 2, v, 0))),
        out_shape=jax.ShapeDtypeStruct((_C, _N), jnp.float32),
        scratch_shapes=[
            pltpu.VMEM((_N, _N), jnp.bfloat16),
            pltpu.VMEM((1, _N), jnp.float32),
            pltpu.VMEM((_F, _N), jnp.bfloat16),
            pltpu.VMEM((_F, _N), jnp.bfloat16),
            pltpu.VMEM((_C, _N), jnp.bfloat16),
        ],
        compiler_params=pltpu.CompilerParams(
            dimension_semantics=("arbitrary", "arbitrary"),
            vmem_limit_bytes=128 * 1024 * 1024,
        ),
    )(hcatT, adj, w0, b0.reshape(-1, 1), w1, b1.reshape(-1, 1),
      w12, b12, W_self2.astype(bf), b2.reshape(-1, 1))
    return outT.T
